# Initial kernel scaffold; baseline (speedup 1.0000x reference)
import functools

import jax
import jax.numpy as jnp
from jax import lax
from jax.experimental import pallas as pl
from jax.experimental.pallas import tpu as pltpu
from jax.experimental.pallas import tpu_sc as plsc

NC = 2
NS = 16
L = 16


def _proj1_body(x_ref, wp_ref, bp_ref, w1_ref, s1_ref, xprow_ref, tdst_ref):
    h0 = jnp.dot(x_ref[...], wp_ref[...], preferred_element_type=jnp.float32)
    h0 = h0 + bp_ref[...]
    xp = jnp.dot(h0, w1_ref[...], preferred_element_type=jnp.float32)
    sc = jnp.dot(xp, s1_ref[...], preferred_element_type=jnp.float32)
    xprow_ref[:, 0:512] = xp
    xprow_ref[:, 512:528] = sc[:, 0:16]
    tdst_ref[...] = sc[:, 16:32]


def _proj2_body(acc_ref, w2_ref, s2_ref, b1_ref, xprow_ref, tdst_ref):
    a = acc_ref[...]
    zs = []
    for h in range(4):
        den = a[:, 512 + h:513 + h] + 1e-16
        zs.append(a[:, h * 128:(h + 1) * 128] / den)
    z = jnp.concatenate(zs, axis=1) + b1_ref[...]
    h1 = jnp.where(z > 0, z, jnp.expm1(z))
    xp2 = jnp.dot(h1, w2_ref[...], preferred_element_type=jnp.float32)
    sc = jnp.dot(xp2, s2_ref[...], preferred_element_type=jnp.float32)
    xprow_ref[:, 0:128] = xp2
    xprow_ref[:, 128:144] = sc[:, 0:16]
    tdst_ref[...] = sc[:, 16:32]


def _final_body(acc_ref, b2_ref, out_ref):
    a = acc_ref[...]
    out_ref[...] = a[:, 0:128] / (a[:, 128:129] + 1e-16) + b2_ref[...]


def _make_edge_kernel(n_nodes, n_edges, heads, roww, chunk, npass, cap,
                      scan_b, k_batch):
    npad = npass * NC * chunk
    so = heads * 128
    e_tile = n_edges // NS
    n_stage = e_tile // scan_b
    rows_pt = chunk // NS
    mesh = plsc.VectorSubcoreMesh(core_axis_name="c", subcore_axis_name="s")

    @functools.partial(
        pl.kernel,
        out_type=jax.ShapeDtypeStruct((npad, roww), jnp.float32),
        mesh=mesh,
        scratch_types=dict(
            accum=pltpu.VMEM_SHARED((chunk, roww), jnp.float32),
            estage_s=pltpu.VMEM((scan_b,), jnp.int32),
            estage_d=pltpu.VMEM((scan_b,), jnp.int32),
            sel_s=pltpu.VMEM((cap + 4 * L,), jnp.int32),
            sel_d=pltpu.VMEM((cap + 4 * L,), jnp.int32),
            rowbuf=pltpu.VMEM((k_batch, roww), jnp.float32),
            tdbuf=pltpu.VMEM((k_batch, L), jnp.float32),
            scatidx=pltpu.VMEM((1, k_batch), jnp.int32),
            zbuf=pltpu.VMEM((8, roww), jnp.float32),
        ),
    )
    def edge_kernel(src_hbm, dst_hbm, xprow_hbm, tdst_hbm, out_hbm,
                    accum, estage_s, estage_d, sel_s, sel_d, rowbuf,
                    tdbuf, scatidx, zbuf):
        c = lax.axis_index("c")
        s = lax.axis_index("s")
        ebase = s * e_tile
        zvec = jnp.zeros((L,), jnp.float32)
        lane = lax.broadcasted_iota(jnp.int32, (L,), 0)

        @pl.loop(0, 8)
        def _zr(r):
            @pl.loop(0, roww // L)
            def _zc(cc):
                zbuf[r, pl.ds(cc * L, L)] = zvec

        @pl.loop(0, npass)
        def _pass(p):
            base = (p * NC + c) * chunk

            @pl.loop(0, rows_pt // 8)
            def _z(i):
                pltpu.sync_copy(zbuf, accum.at[pl.ds(s * rows_pt + i * 8, 8)])

            plsc.subcore_barrier()

            def _stage(st, cnt):
                pltpu.sync_copy(src_hbm.at[pl.ds(ebase + st * scan_b, scan_b)],
                                estage_s)
                pltpu.sync_copy(dst_hbm.at[pl.ds(ebase + st * scan_b, scan_b)],
                                estage_d)

                def _vec(i, cnt):
                    sv = estage_s[pl.ds(i * L, L)]
                    dv = estage_d[pl.ds(i * L, L)]
                    m = (dv >= base) & (dv < base + chunk)
                    plsc.store_compressed(sel_s.at[pl.ds(cnt, L)], sv, m)
                    plsc.store_compressed(sel_d.at[pl.ds(cnt, L)], dv, m)
                    return cnt + jnp.sum(m.astype(jnp.int32))

                return lax.fori_loop(0, scan_b // L, _vec, cnt)

            cnt = lax.fori_loop(0, n_stage, _stage, jnp.int32(0))

            for jj in range(4):
                sel_s[pl.ds(cnt + jj * L, L)] = jnp.zeros((L,), jnp.int32)
                sel_d[pl.ds(cnt + jj * L, L)] = jnp.full((L,), base, jnp.int32)

            nbatch = (cnt + (k_batch - 1)) // k_batch

            @pl.loop(0, nbatch)
            def _batch(b):
                boff = b * k_batch
                pltpu.sync_copy(xprow_hbm.at[sel_s.at[pl.ds(boff, k_batch)]],
                                rowbuf)
                pltpu.sync_copy(tdst_hbm.at[sel_d.at[pl.ds(boff, k_batch)]],
                                tdbuf)

                @pl.loop(0, k_batch // L)
                def _si(i):
                    v = sel_d[pl.ds(boff + i * L, L)] - base
                    scatidx[0, pl.ds(i * L, L)] = v

                @pl.loop(0, k_batch)
                def _edge(j):
                    a = rowbuf[j, pl.ds(so, L)] + tdbuf[j, :]
                    al = jnp.where(a >= 0, a, 0.2 * a)
                    w = jnp.exp(al)
                    ok = (boff + j) < cnt
                    w = jnp.where(ok & (lane < heads), w, 0.0)
                    rowbuf[j, pl.ds(so, L)] = w
                    for h in range(heads):
                        wh = rowbuf[j, so + h]
                        for v in range(8):
                            col = h * 128 + v * L
                            rowbuf[j, pl.ds(col, L)] = (
                                rowbuf[j, pl.ds(col, L)] * wh)

                pltpu.sync_copy(rowbuf, accum.at[scatidx.at[0]], add=True)

            plsc.subcore_barrier()

            @pl.loop(0, rows_pt // 8)
            def _f(i):
                r0 = s * rows_pt + i * 8
                pltpu.sync_copy(accum.at[pl.ds(r0, 8)],
                                out_hbm.at[pl.ds(base + r0, 8)])

            plsc.subcore_barrier()

    return edge_kernel


def kernel(x, edge_index, Wp, bp, W1, as1, ad1, b1, W2, as2, ad2, b2):
    n = x.shape[0]
    e = edge_index.shape[1]
    src = edge_index[0].astype(jnp.int32)
    dst = edge_index[1].astype(jnp.int32)

    S1 = jnp.zeros((512, 32), jnp.float32)
    for h in range(4):
        S1 = S1.at[h * 128:(h + 1) * 128, h].set(as1[h])
        S1 = S1.at[h * 128:(h + 1) * 128, 16 + h].set(ad1[h])
    S2 = jnp.zeros((128, 32), jnp.float32)
    S2 = S2.at[:, 0].set(as2[0])
    S2 = S2.at[:, 16].set(ad2[0])

    blk = 512
    g1 = pl.cdiv(n, blk)

    xprow1, tdst1 = pl.pallas_call(
        _proj1_body,
        grid=(g1,),
        in_specs=[
            pl.BlockSpec((blk, 768), lambda i: (i, 0)),
            pl.BlockSpec((768, 128), lambda i: (0, 0)),
            pl.BlockSpec((128,), lambda i: (0,)),
            pl.BlockSpec((128, 512), lambda i: (0, 0)),
            pl.BlockSpec((512, 32), lambda i: (0, 0)),
        ],
        out_specs=[
            pl.BlockSpec((blk, 528), lambda i: (i, 0)),
            pl.BlockSpec((blk, 16), lambda i: (i, 0)),
        ],
        out_shape=[
            jax.ShapeDtypeStruct((n, 528), jnp.float32),
            jax.ShapeDtypeStruct((n, 16), jnp.float32),
        ],
    )(x, Wp, bp, W1, S1)

    ek1 = _make_edge_kernel(n, e, heads=4, roww=528, chunk=3584, npass=7,
                            cap=9216, scan_b=2000, k_batch=64)
    acc1 = ek1(src, dst, xprow1, tdst1)

    xprow2, tdst2 = pl.pallas_call(
        _proj2_body,
        grid=(g1,),
        in_specs=[
            pl.BlockSpec((blk, 528), lambda i: (i, 0)),
            pl.BlockSpec((512, 128), lambda i: (0, 0)),
            pl.BlockSpec((128, 32), lambda i: (0, 0)),
            pl.BlockSpec((512,), lambda i: (0,)),
        ],
        out_specs=[
            pl.BlockSpec((blk, 144), lambda i: (i, 0)),
            pl.BlockSpec((blk, 16), lambda i: (i, 0)),
        ],
        out_shape=[
            jax.ShapeDtypeStruct((n, 144), jnp.float32),
            jax.ShapeDtypeStruct((n, 16), jnp.float32),
        ],
    )(acc1[:n], W2, S2, b1)

    ek2 = _make_edge_kernel(n, e, heads=1, roww=144, chunk=12800, npass=2,
                            cap=28672, scan_b=2000, k_batch=64)
    acc2 = ek2(src, dst, xprow2, tdst2)

    out = pl.pallas_call(
        _final_body,
        grid=(g1,),
        in_specs=[
            pl.BlockSpec((blk, 144), lambda i: (i, 0)),
            pl.BlockSpec((128,), lambda i: (0,)),
        ],
        out_specs=pl.BlockSpec((blk, 128), lambda i: (i, 0)),
        out_shape=jax.ShapeDtypeStruct((n, 128), jnp.float32),
    )(acc2[:n], b2)

    return out


# speedup vs baseline: 8.4693x; 8.4693x over previous
import functools

import jax
import jax.numpy as jnp
from jax import lax
from jax.experimental import pallas as pl
from jax.experimental.pallas import tpu as pltpu
from jax.experimental.pallas import tpu_sc as plsc

NC = 2
NS = 16
NW = NC * NS
L = 16
NPR = 50176
OWN = NPR // NW
E_PAD = 819200
ET = E_PAD // NW
SCAN = 1600
C1 = 1408
STG = 128


def _proj1_body(x_ref, wp_ref, bp_ref, w1_ref, s1_ref, d1_ref,
                xprow_ref, tsc_ref):
    h0 = jnp.dot(x_ref[...], wp_ref[...], preferred_element_type=jnp.float32)
    h0 = h0 + bp_ref[...]
    xp = jnp.dot(h0, w1_ref[...], preferred_element_type=jnp.float32)
    xprow_ref[:, 0:512] = xp
    xprow_ref[:, 512:640] = jnp.dot(xp, s1_ref[...],
                                    preferred_element_type=jnp.float32)
    tsc_ref[...] = jnp.dot(xp, d1_ref[...],
                           preferred_element_type=jnp.float32)


def _proj2_body(a_ref, w2_ref, s2_ref, d2_ref, b1_ref, xprow_ref, tsc_ref):
    a = a_ref[...]
    zs = []
    for h in range(4):
        den = a[:, 512 + h:513 + h] + 1e-16
        zs.append(a[:, h * 128:(h + 1) * 128] / den)
    z = jnp.concatenate(zs, axis=1) + b1_ref[...]
    h1 = jnp.where(z > 0, z, jnp.exp(jnp.minimum(z, 0.0)) - 1.0)
    xp2 = jnp.dot(h1, w2_ref[...], preferred_element_type=jnp.float32)
    xprow_ref[:, 0:128] = xp2
    xprow_ref[:, 128:256] = jnp.dot(xp2, s2_ref[...],
                                    preferred_element_type=jnp.float32)
    tsc_ref[...] = jnp.dot(xp2, d2_ref[...],
                           preferred_element_type=jnp.float32)


def _final_body(a_ref, b2_ref, out_ref):
    a = a_ref[...]
    out_ref[...] = a[:, 0:128] / (a[:, 128:129] + 1e-16) + b2_ref[...]


def _make_bin_kernel(e_real):
    mesh = plsc.VectorSubcoreMesh(core_axis_name="c", subcore_axis_name="s")

    @functools.partial(
        pl.kernel,
        out_type=[
            jax.ShapeDtypeStruct((NW * NW * C1,), jnp.int32),
            jax.ShapeDtypeStruct((NW * NW * C1,), jnp.int32),
            jax.ShapeDtypeStruct((NW * NW,), jnp.int32),
        ],
        mesh=mesh,
        compiler_params=pltpu.CompilerParams(needs_layout_passes=False),
        scratch_types=dict(
            estage_s=pltpu.VMEM((SCAN,), jnp.int32),
            estage_d=pltpu.VMEM((SCAN,), jnp.int32),
            stg_s=pltpu.VMEM((NW * STG,), jnp.int32),
            stg_d=pltpu.VMEM((NW * STG,), jnp.int32),
            cntv=pltpu.VMEM((1, NW), jnp.int32),
            cnt_sm=pltpu.SMEM((NW,), jnp.int32),
        ),
    )
    def bin_kernel(src_hbm, dst_hbm, bsrc_hbm, bdst_hbm, counts_hbm,
                   estage_s, estage_d, stg_s, stg_d, cntv, cnt_sm):
        c = lax.axis_index("c")
        s = lax.axis_index("s")
        w = s * NC + c
        ebase = w * ET
        lane = lax.broadcasted_iota(jnp.int32, (L,), 0)

        @pl.loop(0, NW)
        def _z(o):
            cnt_sm[o] = 0

        @pl.loop(0, ET // SCAN)
        def _stage(st):
            eoff = pl.multiple_of(ebase + st * SCAN, 8)
            pltpu.sync_copy(src_hbm.at[pl.ds(eoff, SCAN)], estage_s)
            pltpu.sync_copy(dst_hbm.at[pl.ds(eoff, SCAN)], estage_d)

            @pl.loop(0, SCAN // L)
            def _vec(i):
                sv = estage_s[pl.ds(i * L, L)]
                dv = estage_d[pl.ds(i * L, L)]
                eg = ebase + st * SCAN + i * L + lane
                validv = (eg < e_real).astype(jnp.int32)
                ov = (dv * 42800) >> 26
                ov = ov - ((ov * OWN) > dv).astype(jnp.int32)
                for jj in range(L):
                    @pl.when(validv[jj] != 0)
                    def _append():
                        o = ov[jj]
                        cnt = cnt_sm[o]
                        slot = lax.rem(cnt, STG)
                        pos = o * STG + slot
                        plsc.store_scatter(
                            stg_s, [jnp.full((L,), pos, jnp.int32)],
                            jnp.full((L,), sv[jj], jnp.int32),
                            mask=lane == 0)
                        plsc.store_scatter(
                            stg_d, [jnp.full((L,), pos, jnp.int32)],
                            jnp.full((L,), dv[jj], jnp.int32),
                            mask=lane == 0)
                        cnt_sm[o] = cnt + 1

                        @pl.when(slot == STG - 1)
                        def _flush():
                            dsto = pl.multiple_of(
                                (o * NW + w) * C1 + cnt - (STG - 1), 8)
                            srco = pl.multiple_of(o * STG, 8)
                            pltpu.sync_copy(stg_s.at[pl.ds(srco, STG)],
                                            bsrc_hbm.at[pl.ds(dsto, STG)])
                            pltpu.sync_copy(stg_d.at[pl.ds(srco, STG)],
                                            bdst_hbm.at[pl.ds(dsto, STG)])

        @pl.loop(0, NW)
        def _tail(o):
            cnt = cnt_sm[o]
            rem = lax.rem(cnt, STG)

            @pl.when(rem > 0)
            def _flush():
                dsto = pl.multiple_of((o * NW + w) * C1 + cnt - rem, 8)
                srco = pl.multiple_of(o * STG, 8)
                pltpu.sync_copy(stg_s.at[pl.ds(srco, STG)],
                                bsrc_hbm.at[pl.ds(dsto, STG)])
                pltpu.sync_copy(stg_d.at[pl.ds(srco, STG)],
                                bdst_hbm.at[pl.ds(dsto, STG)])

            plsc.store_scatter(cntv.at[0], [jnp.full((L,), o, jnp.int32)],
                               jnp.full((L,), cnt, jnp.int32),
                               mask=lane == 0)

        pltpu.sync_copy(cntv.at[0],
                        counts_hbm.at[pl.ds(pl.multiple_of(w * NW, 8), NW)])

    return bin_kernel


def _make_edge_kernel(heads, roww, sub, selcap):
    so = heads * 128
    nsub = -(-OWN // sub)
    hg = NW // 2
    mesh = plsc.VectorSubcoreMesh(core_axis_name="c", subcore_axis_name="s")

    @functools.partial(
        pl.kernel,
        out_type=jax.ShapeDtypeStruct((NPR, roww), jnp.float32),
        mesh=mesh,
        compiler_params=pltpu.CompilerParams(needs_layout_passes=False),
        scratch_types=dict(
            cbuf=pltpu.VMEM((NW * NW + L,), jnp.int32),
            ebuf_s=pltpu.VMEM((hg * C1,), jnp.int32),
            ebuf_d=pltpu.VMEM((hg * C1,), jnp.int32),
            sel_s=pltpu.VMEM((selcap + 4 * L,), jnp.int32),
            sel_d=pltpu.VMEM((selcap + 4 * L,), jnp.int32),
            accum=pltpu.VMEM((sub, roww), jnp.float32),
            rowbuf=pltpu.VMEM((L, roww), jnp.float32),
            tdbuf=pltpu.VMEM((L, 128), jnp.float32),
        ),
    )
    def edge_kernel(bsrc_hbm, bdst_hbm, counts_hbm, xprow_hbm, tsc_hbm,
                    out_hbm, cbuf, ebuf_s, ebuf_d, sel_s, sel_d, accum,
                    rowbuf, tdbuf):
        c = lax.axis_index("c")
        s = lax.axis_index("s")
        w = s * NC + c
        lane = lax.broadcasted_iota(jnp.int32, (L,), 0)
        zvec = jnp.zeros((L,), jnp.float32)

        pltpu.sync_copy(counts_hbm, cbuf.at[pl.ds(0, NW * NW)])

        @pl.loop(0, nsub)
        def _sub(sb):
            sub0 = w * OWN + sb * sub
            sublen = jnp.minimum(sub, OWN - sb * sub)

            @pl.loop(0, sub)
            def _zr(r):
                @pl.loop(0, roww // L)
                def _zc(cc):
                    accum[r, pl.ds(cc * L, L)] = zvec

            @pl.loop(0, 2)
            def _grp(g):
                goff = pl.multiple_of((w * NW + g * hg) * C1, 8)
                pltpu.sync_copy(bsrc_hbm.at[pl.ds(goff, hg * C1)], ebuf_s)
                pltpu.sync_copy(bdst_hbm.at[pl.ds(goff, hg * C1)], ebuf_d)

                def _sender(r, nsel):
                    cnt = cbuf[pl.ds((g * hg + r) * NW + w, L)][0]

                    def _vec(i, nsel):
                        dv = ebuf_d[pl.ds(r * C1 + i * L, L)]
                        sv = ebuf_s[pl.ds(r * C1 + i * L, L)]
                        m = ((i * L + lane) < cnt) & (dv >= sub0) \
                            & (dv < sub0 + sublen)
                        pref = plsc.cumsum(m.astype(jnp.int32))
                        pos = nsel + pref - 1
                        plsc.store_scatter(sel_s, [pos], sv, mask=m)
                        plsc.store_scatter(sel_d, [pos], dv, mask=m)
                        return nsel + pref[L - 1]

                    return lax.fori_loop(0, (cnt + L - 1) // L, _vec, nsel)

                nsel = lax.fori_loop(0, hg, _sender, jnp.int32(0))

                sel_s[pl.ds(nsel, L)] = jnp.zeros((L,), jnp.int32)
                sel_d[pl.ds(nsel, L)] = jnp.full((L,), sub0, jnp.int32)

                @pl.loop(0, (nsel + L - 1) // L)
                def _batch(b):
                    boff = pl.multiple_of(b * L, 8)
                    pltpu.sync_copy(
                        xprow_hbm.at[sel_s.at[pl.ds(boff, L)]], rowbuf)
                    pltpu.sync_copy(
                        tsc_hbm.at[sel_d.at[pl.ds(boff, L)]], tdbuf)
                    rlv = sel_d[pl.ds(boff, L)] - sub0
                    for jj in range(L):
                        a = rowbuf[jj, pl.ds(so, L)] + tdbuf[jj, pl.ds(0, L)]
                        al = jnp.where(a >= 0, a, 0.2 * a)
                        wv = jnp.exp(al)
                        ok = (boff + jj) < nsel
                        wv = jnp.where(ok & (lane < heads), wv, 0.0)
                        rl = rlv[jj]
                        rsp = jnp.full((L,), rl, jnp.int32)
                        plsc.addupdate_scatter(
                            accum, [rsp, so + lane], wv)
                        for h in range(heads):
                            wh = wv[h]
                            for v in range(8):
                                col = h * 128 + v * L
                                xv = rowbuf[jj, pl.ds(col, L)]
                                plsc.addupdate_scatter(
                                    accum, [rsp, col + lane], xv * wh)

            @pl.loop(0, sublen // 32)
            def _f(i):
                pltpu.sync_copy(accum.at[pl.ds(i * 32, 32)],
                                out_hbm.at[pl.ds(sub0 + i * 32, 32)])

    return edge_kernel


def kernel(x, edge_index, Wp, bp, W1, as1, ad1, b1, W2, as2, ad2, b2):
    n = x.shape[0]
    e = edge_index.shape[1]
    src = edge_index[0].astype(jnp.int32)
    dst = edge_index[1].astype(jnp.int32)
    pad = jnp.zeros((E_PAD - e,), jnp.int32)
    src = jnp.concatenate([src, pad])
    dst = jnp.concatenate([dst, pad])

    S1 = jnp.zeros((512, 128), jnp.float32)
    D1 = jnp.zeros((512, 128), jnp.float32)
    for h in range(4):
        S1 = S1.at[h * 128:(h + 1) * 128, h].set(as1[h])
        D1 = D1.at[h * 128:(h + 1) * 128, h].set(ad1[h])
    S2 = jnp.zeros((128, 128), jnp.float32)
    S2 = S2.at[:, 0].set(as2[0])
    D2 = jnp.zeros((128, 128), jnp.float32)
    D2 = D2.at[:, 0].set(ad2[0])

    blk = 512
    g1 = pl.cdiv(n, blk)

    bsrc, bdst, counts = _make_bin_kernel(e)(src, dst)

    xprow1, tsc1 = pl.pallas_call(
        _proj1_body,
        grid=(g1,),
        in_specs=[
            pl.BlockSpec((blk, 768), lambda i: (i, 0)),
            pl.BlockSpec((768, 128), lambda i: (0, 0)),
            pl.BlockSpec((128,), lambda i: (0,)),
            pl.BlockSpec((128, 512), lambda i: (0, 0)),
            pl.BlockSpec((512, 128), lambda i: (0, 0)),
            pl.BlockSpec((512, 128), lambda i: (0, 0)),
        ],
        out_specs=[
            pl.BlockSpec((blk, 640), lambda i: (i, 0)),
            pl.BlockSpec((blk, 128), lambda i: (i, 0)),
        ],
        out_shape=[
            jax.ShapeDtypeStruct((n, 640), jnp.float32),
            jax.ShapeDtypeStruct((n, 128), jnp.float32),
        ],
    )(x, Wp, bp, W1, S1, D1)

    ek1 = _make_edge_kernel(heads=4, roww=640, sub=64, selcap=832)
    acc1 = ek1(bsrc, bdst, counts, xprow1, tsc1)

    xprow2, tsc2 = pl.pallas_call(
        _proj2_body,
        grid=(g1,),
        in_specs=[
            pl.BlockSpec((blk, 640), lambda i: (i, 0)),
            pl.BlockSpec((512, 128), lambda i: (0, 0)),
            pl.BlockSpec((128, 128), lambda i: (0, 0)),
            pl.BlockSpec((128, 128), lambda i: (0, 0)),
            pl.BlockSpec((512,), lambda i: (0,)),
        ],
        out_specs=[
            pl.BlockSpec((blk, 256), lambda i: (i, 0)),
            pl.BlockSpec((blk, 128), lambda i: (i, 0)),
        ],
        out_shape=[
            jax.ShapeDtypeStruct((n, 256), jnp.float32),
            jax.ShapeDtypeStruct((n, 128), jnp.float32),
        ],
    )(acc1, W2, S2, D2, b1)

    ek2 = _make_edge_kernel(heads=1, roww=256, sub=128, selcap=1600)
    acc2 = ek2(bsrc, bdst, counts, xprow2, tsc2)

    out = pl.pallas_call(
        _final_body,
        grid=(g1,),
        in_specs=[
            pl.BlockSpec((blk, 256), lambda i: (i, 0)),
            pl.BlockSpec((128,), lambda i: (0,)),
        ],
        out_specs=pl.BlockSpec((blk, 128), lambda i: (i, 0)),
        out_shape=jax.ShapeDtypeStruct((n, 128), jnp.float32),
    )(acc2, b2)

    return out
